# baseline (speedup 1.0000x reference)
import functools

import jax
import jax.numpy as jnp
from jax import lax
from jax.experimental import pallas as pl
from jax.experimental.pallas import tpu as pltpu
from jax.experimental.pallas import tpu_sc as plsc

_LANES = 128
_K = 48


def _pad_table(embT, V, E, C=512):
    def body(t_ref, e_ref, o_ref):
        t = lax.dot_general(
            t_ref[...], e_ref[...], (((0,), (0,)), ((), ())),
            precision=lax.Precision.HIGHEST,
            preferred_element_type=jnp.float32,
        )
        o_ref[:, pl.ds(0, E)] = t
        o_ref[:, pl.ds(E, _LANES - E)] = jnp.zeros((C, _LANES - E), jnp.float32)

    return pl.pallas_call(
        body,
        grid=(pl.cdiv(V, C),),
        in_specs=[
            pl.BlockSpec((E, C), lambda j: (0, j)),
            pl.BlockSpec((E, E), lambda j: (0, 0)),
        ],
        out_specs=pl.BlockSpec((C, _LANES), lambda j: (j, 0)),
        out_shape=jax.ShapeDtypeStruct((V, _LANES), jnp.float32),
    )(embT, jnp.eye(E, dtype=jnp.float32))


def _sc_mean(emb128, inT, B, L, NW, BPW):
    IPW = BPW * L

    mesh = plsc.VectorSubcoreMesh(core_axis_name="c", subcore_axis_name="s")

    @functools.partial(
        pl.kernel,
        out_type=jax.ShapeDtypeStruct((B, _K), jnp.float32),
        mesh=mesh,
        scratch_types=[
            pltpu.VMEM((L * B,), jnp.int32),
            pltpu.VMEM((IPW, _LANES), jnp.float32),
            pltpu.VMEM((BPW, _K), jnp.float32),
            pltpu.SemaphoreType.DMA,
            pltpu.SemaphoreType.DMA,
        ],
    )
    def run(emb_hbm, in_hbm, out_hbm, iv, rows_v, acc_v, sem, sem2):
        wid = lax.axis_index("s") * 2 + lax.axis_index("c")
        base = wid * BPW
        stages = [
            pltpu.async_copy(in_hbm.at[j], iv.at[pl.ds(j * B, B)], sem2)
            for j in range(L)
        ]
        for s in stages:
            s.wait()
        copies = [
            pltpu.async_copy(emb_hbm.at[iv.at[pl.ds(j * B + base, BPW)]],
                             rows_v.at[pl.ds(j * BPW, BPW)], sem)
            for j in range(L)
        ]
        for c in copies:
            c.wait()

        inv_l = 1.0 / L

        def outer(bi, carry):
            a0 = rows_v[bi, pl.ds(0, 16)]
            a1 = rows_v[bi, pl.ds(16, 16)]
            for j in range(1, L):
                r = j * BPW + bi
                a0 = a0 + rows_v[r, pl.ds(0, 16)]
                a1 = a1 + rows_v[r, pl.ds(16, 16)]
            acc_v[bi, pl.ds(0, 16)] = a0 * inv_l
            acc_v[bi, pl.ds(16, 16)] = a1 * inv_l
            return carry

        lax.fori_loop(0, BPW, outer, 0)
        pltpu.sync_copy(acc_v, out_hbm.at[pl.ds(base, BPW)])

    return run(emb128, inT)


def _project_t(wtb, means, E, V, B, VB):
    def body(w_ref, m_ref, o_ref):
        m = m_ref[...]
        lane = lax.broadcasted_iota(jnp.int32, (B, _K), 1)
        m = jnp.where(lane == E, 1.0, jnp.where(lane > E, 0.0, m))
        o_ref[...] = lax.dot_general(
            w_ref[...], m,
            (((0,), (1,)), ((), ())),
            preferred_element_type=jnp.float32,
        )

    return pl.pallas_call(
        body,
        grid=(pl.cdiv(V, VB),),
        in_specs=[
            pl.BlockSpec((_K, VB), lambda j: (0, j)),
            pl.BlockSpec((B, _K), lambda j: (0, 0)),
        ],
        out_specs=pl.BlockSpec((VB, B), lambda j: (j, 0)),
        out_shape=jax.ShapeDtypeStruct((V, B), jnp.float32),
    )(wtb, means)


def kernel(inputs, emb, W, b):
    B, S = inputs.shape
    V, E = emb.shape
    L = S - 1
    NW = 32
    BPW = B // NW

    emb128 = _pad_table(emb.T, V, E)
    inT = inputs.T.astype(jnp.int32)
    means = _sc_mean(emb128, inT, B, L, NW, BPW)
    wtb = jnp.concatenate(
        [W.T, b[None, :], jnp.zeros((_K - E - 1, V), jnp.float32)], axis=0
    )
    ot = _project_t(wtb, means, E, V, B, VB=2048)
    return ot.T


# speedup vs baseline: 1.4261x; 1.4261x over previous
import functools

import jax
import jax.numpy as jnp
from jax import lax
from jax.experimental import pallas as pl
from jax.experimental.pallas import tpu as pltpu
from jax.experimental.pallas import tpu_sc as plsc

_LANES = 128
_K = 48


def _sc_mean(emb128, inT, B, L, NW, BPW):
    IPW = BPW * L

    mesh = plsc.VectorSubcoreMesh(core_axis_name="c", subcore_axis_name="s")

    @functools.partial(
        pl.kernel,
        out_type=jax.ShapeDtypeStruct((B, _K), jnp.float32),
        mesh=mesh,
        scratch_types=[
            pltpu.VMEM((L * B,), jnp.int32),
            pltpu.VMEM((IPW, _LANES), jnp.float32),
            pltpu.VMEM((BPW, _K), jnp.float32),
            pltpu.SemaphoreType.DMA,
            pltpu.SemaphoreType.DMA,
        ],
    )
    def run(emb_hbm, in_hbm, out_hbm, iv, rows_v, acc_v, sem, sem2):
        wid = lax.axis_index("s") * 2 + lax.axis_index("c")
        base = wid * BPW
        stages = [
            pltpu.async_copy(in_hbm.at[j], iv.at[pl.ds(j * B, B)], sem2)
            for j in range(L)
        ]
        for s in stages:
            s.wait()
        copies = [
            pltpu.async_copy(emb_hbm.at[iv.at[pl.ds(j * B + base, BPW)]],
                             rows_v.at[pl.ds(j * BPW, BPW)], sem)
            for j in range(L)
        ]
        for c in copies:
            c.wait()

        inv_l = 1.0 / L

        def outer(bi, carry):
            a0 = rows_v[bi, pl.ds(0, 16)]
            a1 = rows_v[bi, pl.ds(16, 16)]
            for j in range(1, L):
                r = j * BPW + bi
                a0 = a0 + rows_v[r, pl.ds(0, 16)]
                a1 = a1 + rows_v[r, pl.ds(16, 16)]
            acc_v[bi, pl.ds(0, 16)] = a0 * inv_l
            acc_v[bi, pl.ds(16, 16)] = a1 * inv_l
            return carry

        lax.fori_loop(0, BPW, outer, 0)
        pltpu.sync_copy(acc_v, out_hbm.at[pl.ds(base, BPW)])

    return run(emb128, inT)


def _project_t(wtb, means, E, V, B, VB):
    def body(w_ref, m_ref, o_ref):
        m = m_ref[...]
        lane = lax.broadcasted_iota(jnp.int32, (B, _K), 1)
        m = jnp.where(lane == E, 1.0, jnp.where(lane > E, 0.0, m))
        o_ref[...] = lax.dot_general(
            w_ref[...], m,
            (((0,), (1,)), ((), ())),
            preferred_element_type=jnp.float32,
        )

    return pl.pallas_call(
        body,
        grid=(pl.cdiv(V, VB),),
        in_specs=[
            pl.BlockSpec((_K, VB), lambda j: (0, j)),
            pl.BlockSpec((B, _K), lambda j: (0, 0)),
        ],
        out_specs=pl.BlockSpec((VB, B), lambda j: (j, 0)),
        out_shape=jax.ShapeDtypeStruct((V, B), jnp.float32),
    )(wtb, means)


def kernel(inputs, emb, W, b):
    B, S = inputs.shape
    V, E = emb.shape
    L = S - 1
    NW = 32
    BPW = B // NW

    emb128 = jnp.pad(emb, ((0, 0), (0, _LANES - E)))
    inT = inputs.T.astype(jnp.int32)
    means = _sc_mean(emb128, inT, B, L, NW, BPW)
    wtb = jnp.concatenate(
        [W.T, b[None, :], jnp.zeros((_K - E - 1, V), jnp.float32)], axis=0
    )
    ot = _project_t(wtb, means, E, V, B, VB=3072)
    return ot.T
